# baseline (speedup 1.0000x reference)
import functools

import jax
import jax.numpy as jnp
from jax import lax
from jax.experimental import pallas as pl
from jax.experimental.pallas import tpu as pltpu
from jax.experimental.pallas import tpu_sc as plsc

_B, _N, _E = 8, 10000, 320000
_K = 2000
_EPT = _E // 4
_CHUNKS = _EPT // _K


def _prep_kernel(fp, fl, fti, ftv, v, th, lf, tl, a_out, c_out, s_out):
    step = pl.program_id(0)

    @pl.when(step == 0)
    def _():
        p = jnp.clip(fp[...], 1e-7, 1.0 - 1e-7)
        t = fl[...]
        bce = jnp.sum(t * jnp.log(p) + (1.0 - t) * jnp.log(1.0 - p))
        mask = t > 0.5
        cnt = jnp.sum(mask.astype(jnp.float32))
        ft = ftv[...][:, :1]
        sq = jnp.sum(jnp.where(mask, (fti[...] - ft) ** 2, 0.0))
        vv = v[...]
        lo = jnp.maximum(0.95 - vv, 0.0)
        hi = jnp.maximum(vv - 1.05, 0.0)
        stab = jnp.sum(lo * lo + hi * hi)
        a_out[...] = vv * jnp.cos(th[...])
        c_out[...] = vv * jnp.sin(th[...])
        s_out[0, 0] = bce
        s_out[0, 1] = cnt
        s_out[0, 2] = sq
        s_out[0, 3] = stab
        s_out[0, 4] = 0.0

    viol = jnp.maximum(lf[...] - tl[...], 0.0)
    s_out[0, 4] = s_out[0, 4] + jnp.sum(viol * viol)


def _final_kernel(pp, pinj, o):
    pc = pp[0] + pp[1] + pp[2] + pp[3]
    d = pc - pinj[...]
    o[0, 0] = jnp.sum(d * d)


def _pf_body(a_hbm, c_hbm, big_hbm, out_hbm,
             a_tab, c_tab,
             src0, dst0, g0, b0,
             src1, dst1, g1, b1,
             acc_p, ld0, ld1):
    c_id = lax.axis_index("c")
    s_id = lax.axis_index("s")
    lb = s_id // 4
    q = s_id % 4
    b_glob = c_id * 4 + lb

    pltpu.async_copy(a_hbm.at[pl.ds(b_glob * _N, _N)], a_tab, ld0)
    pltpu.async_copy(c_hbm.at[pl.ds(b_glob * _N, _N)], c_tab, ld1)
    zero16 = jnp.zeros((16,), jnp.float32)

    @plsc.parallel_loop(0, _N // 16, 1, unroll=4)
    def zloop(i):
        acc_p[pl.ds(i * 16, 16)] = zero16

    pltpu.make_async_copy(a_hbm.at[pl.ds(0, _N)], a_tab, ld0).wait()
    pltpu.make_async_copy(c_hbm.at[pl.ds(0, _N)], c_tab, ld1).wait()

    bufs = ((src0, dst0, g0, b0, ld0),
            (src1, dst1, g1, b1, ld1))

    def issue_loads(t, bs):
        src_v, dst_v, g_v, b_v, ld = bs
        off = q * _EPT + t * _K
        pltpu.async_copy(big_hbm.at[pl.ds(off, _K)], src_v, ld)
        pltpu.async_copy(big_hbm.at[pl.ds(_E + off, _K)], dst_v, ld)
        goff = 2 * _E + b_glob * _E + off
        pltpu.async_copy(big_hbm.at[pl.ds(goff, _K)], g_v, ld)
        pltpu.async_copy(big_hbm.at[pl.ds(goff + _B * _E, _K)], b_v, ld)

    def wait_loads(bs):
        src_v, dst_v, g_v, b_v, ld = bs
        pltpu.make_async_copy(big_hbm.at[pl.ds(0, _K)], src_v, ld).wait()
        pltpu.make_async_copy(big_hbm.at[pl.ds(0, _K)], dst_v, ld).wait()
        pltpu.make_async_copy(big_hbm.at[pl.ds(0, _K)], g_v, ld).wait()
        pltpu.make_async_copy(big_hbm.at[pl.ds(0, _K)], b_v, ld).wait()

    def compute(bs):
        src_v, dst_v, g_v, b_v, _ = bs

        @plsc.parallel_loop(0, _K // 16, 1, unroll=4)
        def grp(j):
            base = j * 16
            si = src_v[pl.ds(base, 16)]
            di = dst_v[pl.ds(base, 16)]
            ai = plsc.load_gather(a_tab, [si])
            aj = plsc.load_gather(a_tab, [di])
            ci = plsc.load_gather(c_tab, [si])
            cj = plsc.load_gather(c_tab, [di])
            g = plsc.bitcast(g_v[pl.ds(base, 16)], jnp.float32)
            bb = plsc.bitcast(b_v[pl.ds(base, 16)], jnp.float32)
            p = g * (ai * aj + ci * cj) + bb * (ci * aj - ai * cj)
            plsc.addupdate_scatter(acc_p, [si], p)
            plsc.addupdate_scatter(acc_p, [di], -p)

    half = _CHUNKS // 2
    issue_loads(0, bufs[0])
    issue_loads(1, bufs[1])

    def pipe_body(m, carry):
        for par in (0, 1):
            bs = bufs[par]
            wait_loads(bs)
            compute(bs)

            @pl.when(m < half - 1)
            def _():
                issue_loads(2 * (m + 1) + par, bs)
        return carry

    lax.fori_loop(0, half, pipe_body, 0)

    pltpu.sync_copy(acc_p, out_hbm.at[pl.ds((q * _B + b_glob) * _N, _N)])


def _powerflow(a, c, ei, g, b):
    mesh = plsc.VectorSubcoreMesh(core_axis_name="c", subcore_axis_name="s")
    kern = functools.partial(
        pl.kernel,
        mesh=mesh,
        out_type=jax.ShapeDtypeStruct((4 * _B * _N,), jnp.float32),
        compiler_params=pltpu.CompilerParams(needs_layout_passes=False),
        scratch_types=[
            pltpu.VMEM((_N,), jnp.float32),
            pltpu.VMEM((_N,), jnp.float32),
        ] + 2 * [
            pltpu.VMEM((_K,), jnp.int32),
            pltpu.VMEM((_K,), jnp.int32),
            pltpu.VMEM((_K,), jnp.int32),
            pltpu.VMEM((_K,), jnp.int32),
        ] + [
            pltpu.VMEM((_N,), jnp.float32),
            pltpu.SemaphoreType.DMA,
            pltpu.SemaphoreType.DMA,
        ],
    )(_pf_body)
    big = jnp.concatenate([
        ei.reshape(-1),
        lax.bitcast_convert_type(g, jnp.int32).reshape(-1),
        lax.bitcast_convert_type(b, jnp.int32).reshape(-1),
    ])
    return kern(a.reshape(-1), c.reshape(-1), big).reshape(4, _B, _N)


def kernel(failure_probability, failure_label, failure_timing, failure_time,
           voltages, angles, edge_index, conductance, susceptance,
           power_injection, line_flows, thermal_limits):
    v = voltages[..., 0]
    th = angles[..., 0]
    g = conductance[..., 0]
    bsus = susceptance[..., 0]
    pinj = power_injection[..., 0]
    lf = line_flows[..., 0]
    tl = thermal_limits[..., 0]
    ftv = jnp.broadcast_to(failure_time[:, None], (_B, 128))
    ei = edge_index.astype(jnp.int32)

    ec = _E // 10
    a, c, s = pl.pallas_call(
        _prep_kernel,
        grid=(10,),
        in_specs=[
            pl.BlockSpec((_B, _N), lambda i: (0, 0)),
            pl.BlockSpec((_B, _N), lambda i: (0, 0)),
            pl.BlockSpec((_B, _N), lambda i: (0, 0)),
            pl.BlockSpec((_B, 128), lambda i: (0, 0)),
            pl.BlockSpec((_B, _N), lambda i: (0, 0)),
            pl.BlockSpec((_B, _N), lambda i: (0, 0)),
            pl.BlockSpec((_B, ec), lambda i: (0, i)),
            pl.BlockSpec((_B, ec), lambda i: (0, i)),
        ],
        out_specs=[
            pl.BlockSpec((_B, _N), lambda i: (0, 0)),
            pl.BlockSpec((_B, _N), lambda i: (0, 0)),
            pl.BlockSpec((1, 8), lambda i: (0, 0), memory_space=pltpu.SMEM),
        ],
        out_shape=[
            jax.ShapeDtypeStruct((_B, _N), jnp.float32),
            jax.ShapeDtypeStruct((_B, _N), jnp.float32),
            jax.ShapeDtypeStruct((1, 8), jnp.float32),
        ],
    )(failure_probability, failure_label, failure_timing, ftv, v, th, lf, tl)

    pp = _powerflow(a, c, ei, g, bsus)

    pf_sum = pl.pallas_call(
        _final_kernel,
        out_specs=pl.BlockSpec(memory_space=pltpu.SMEM),
        out_shape=jax.ShapeDtypeStruct((1, 1), jnp.float32),
    )(pp, pinj)[0, 0]

    bce_sum = s[0, 0]
    cnt = s[0, 1]
    sq_sum = s[0, 2]
    stab_sum = s[0, 3]
    cap_sum = s[0, 4]

    nbn = jnp.float32(_B * _N)
    l_pred = -bce_sum / nbn
    l_timing = sq_sum / jnp.maximum(cnt, 1.0)
    l_pred = l_pred + jnp.where(cnt > 0, 0.5 * l_timing, 0.0)
    l_pf = pf_sum / nbn
    l_cap = cap_sum / jnp.float32(_B * _E)
    l_stab = stab_sum / nbn
    l_temporal = jnp.float32(0.0)
    l_total = (l_pred + 0.1 * l_pf + 0.05 * l_cap + 0.05 * l_stab
               + 0.02 * l_temporal)
    return (l_total, l_pred, l_pf, l_cap, l_stab, l_temporal)


# speedup vs baseline: 1.3376x; 1.3376x over previous
import functools

import jax
import jax.numpy as jnp
from jax import lax
from jax.experimental import pallas as pl
from jax.experimental.pallas import tpu as pltpu
from jax.experimental.pallas import tpu_sc as plsc

_B, _N, _E = 8, 10000, 320000
_K = 2000
_EPT = _E // 4
_CHUNKS = _EPT // _K


def _prep_kernel(fp, fl, fti, ftv, v, th, lf, tl, a_out, c_out, s_out):
    step = pl.program_id(0)

    @pl.when(step == 0)
    def _():
        p = jnp.clip(fp[...], 1e-7, 1.0 - 1e-7)
        t = fl[...]
        bce = jnp.sum(t * jnp.log(p) + (1.0 - t) * jnp.log(1.0 - p))
        mask = t > 0.5
        cnt = jnp.sum(mask.astype(jnp.float32))
        ft = ftv[...][:, :1]
        sq = jnp.sum(jnp.where(mask, (fti[...] - ft) ** 2, 0.0))
        vv = v[...]
        lo = jnp.maximum(0.95 - vv, 0.0)
        hi = jnp.maximum(vv - 1.05, 0.0)
        stab = jnp.sum(lo * lo + hi * hi)
        a_out[...] = vv * jnp.cos(th[...])
        c_out[...] = vv * jnp.sin(th[...])
        s_out[0, 0] = bce
        s_out[0, 1] = cnt
        s_out[0, 2] = sq
        s_out[0, 3] = stab
        s_out[0, 4] = 0.0

    viol = jnp.maximum(lf[...] - tl[...], 0.0)
    s_out[0, 4] = s_out[0, 4] + jnp.sum(viol * viol)


def _final_kernel(pp, pinj, o):
    pc = pp[0] + pp[1] + pp[2] + pp[3]
    d = pc - pinj[...]
    o[0, 0] = jnp.sum(d * d)


def _pf_body(a_hbm, c_hbm, ei_hbm, g_hbm, b_hbm, out_hbm,
             a_tab, c_tab,
             src0, dst0, g0, b0,
             src1, dst1, g1, b1,
             acc_p, ld0, ld1):
    c_id = lax.axis_index("c")
    s_id = lax.axis_index("s")
    lb = s_id // 4
    q = s_id % 4
    b_glob = c_id * 4 + lb

    pltpu.async_copy(a_hbm.at[pl.ds(b_glob * _N, _N)], a_tab, ld0)
    pltpu.async_copy(c_hbm.at[pl.ds(b_glob * _N, _N)], c_tab, ld1)
    zero16 = jnp.zeros((16,), jnp.float32)

    @plsc.parallel_loop(0, _N // 16, 1, unroll=4)
    def zloop(i):
        acc_p[pl.ds(i * 16, 16)] = zero16

    pltpu.make_async_copy(a_hbm.at[pl.ds(0, _N)], a_tab, ld0).wait()
    pltpu.make_async_copy(c_hbm.at[pl.ds(0, _N)], c_tab, ld1).wait()

    bufs = ((src0, dst0, g0, b0, ld0),
            (src1, dst1, g1, b1, ld1))

    def issue_loads(t, bs):
        src_v, dst_v, g_v, b_v, ld = bs
        off = q * _EPT + t * _K
        pltpu.async_copy(ei_hbm.at[pl.ds(off, _K)], src_v, ld)
        pltpu.async_copy(ei_hbm.at[pl.ds(_E + off, _K)], dst_v, ld)
        pltpu.async_copy(g_hbm.at[pl.ds(b_glob * _E + off, _K)], g_v, ld)
        pltpu.async_copy(b_hbm.at[pl.ds(b_glob * _E + off, _K)], b_v, ld)

    def wait_loads(bs):
        src_v, dst_v, g_v, b_v, ld = bs
        pltpu.make_async_copy(ei_hbm.at[pl.ds(0, _K)], src_v, ld).wait()
        pltpu.make_async_copy(ei_hbm.at[pl.ds(0, _K)], dst_v, ld).wait()
        pltpu.make_async_copy(g_hbm.at[pl.ds(0, _K)], g_v, ld).wait()
        pltpu.make_async_copy(b_hbm.at[pl.ds(0, _K)], b_v, ld).wait()

    def compute(bs):
        src_v, dst_v, g_v, b_v, _ = bs

        @plsc.parallel_loop(0, _K // 16, 1, unroll=4)
        def grp(j):
            base = j * 16
            si = src_v[pl.ds(base, 16)]
            di = dst_v[pl.ds(base, 16)]
            ai = plsc.load_gather(a_tab, [si])
            aj = plsc.load_gather(a_tab, [di])
            ci = plsc.load_gather(c_tab, [si])
            cj = plsc.load_gather(c_tab, [di])
            g = g_v[pl.ds(base, 16)]
            bb = b_v[pl.ds(base, 16)]
            p = g * (ai * aj + ci * cj) + bb * (ci * aj - ai * cj)
            plsc.addupdate_scatter(acc_p, [si], p)
            plsc.addupdate_scatter(acc_p, [di], -p)

    half = _CHUNKS // 2
    issue_loads(0, bufs[0])
    issue_loads(1, bufs[1])

    def pipe_body(m, carry):
        for par in (0, 1):
            bs = bufs[par]
            wait_loads(bs)
            compute(bs)

            @pl.when(m < half - 1)
            def _():
                issue_loads(2 * (m + 1) + par, bs)
        return carry

    lax.fori_loop(0, half, pipe_body, 0)

    pltpu.sync_copy(acc_p, out_hbm.at[pl.ds((q * _B + b_glob) * _N, _N)])


def _powerflow(a, c, ei, g, b):
    mesh = plsc.VectorSubcoreMesh(core_axis_name="c", subcore_axis_name="s")
    kern = functools.partial(
        pl.kernel,
        mesh=mesh,
        out_type=jax.ShapeDtypeStruct((4 * _B * _N,), jnp.float32),
        compiler_params=pltpu.CompilerParams(needs_layout_passes=False),
        scratch_types=[
            pltpu.VMEM((_N,), jnp.float32),
            pltpu.VMEM((_N,), jnp.float32),
        ] + 2 * [
            pltpu.VMEM((_K,), jnp.int32),
            pltpu.VMEM((_K,), jnp.int32),
            pltpu.VMEM((_K,), jnp.float32),
            pltpu.VMEM((_K,), jnp.float32),
        ] + [
            pltpu.VMEM((_N,), jnp.float32),
            pltpu.SemaphoreType.DMA,
            pltpu.SemaphoreType.DMA,
        ],
    )(_pf_body)
    return kern(a.reshape(-1), c.reshape(-1), ei.reshape(-1),
                g.reshape(-1), b.reshape(-1)).reshape(4, _B, _N)


def kernel(failure_probability, failure_label, failure_timing, failure_time,
           voltages, angles, edge_index, conductance, susceptance,
           power_injection, line_flows, thermal_limits):
    v = voltages[..., 0]
    th = angles[..., 0]
    g = conductance[..., 0]
    bsus = susceptance[..., 0]
    pinj = power_injection[..., 0]
    lf = line_flows[..., 0]
    tl = thermal_limits[..., 0]
    ftv = jnp.broadcast_to(failure_time[:, None], (_B, 128))
    ei = edge_index.astype(jnp.int32)

    ec = _E // 10
    a, c, s = pl.pallas_call(
        _prep_kernel,
        grid=(10,),
        in_specs=[
            pl.BlockSpec((_B, _N), lambda i: (0, 0)),
            pl.BlockSpec((_B, _N), lambda i: (0, 0)),
            pl.BlockSpec((_B, _N), lambda i: (0, 0)),
            pl.BlockSpec((_B, 128), lambda i: (0, 0)),
            pl.BlockSpec((_B, _N), lambda i: (0, 0)),
            pl.BlockSpec((_B, _N), lambda i: (0, 0)),
            pl.BlockSpec((_B, ec), lambda i: (0, i)),
            pl.BlockSpec((_B, ec), lambda i: (0, i)),
        ],
        out_specs=[
            pl.BlockSpec((_B, _N), lambda i: (0, 0)),
            pl.BlockSpec((_B, _N), lambda i: (0, 0)),
            pl.BlockSpec((1, 8), lambda i: (0, 0), memory_space=pltpu.SMEM),
        ],
        out_shape=[
            jax.ShapeDtypeStruct((_B, _N), jnp.float32),
            jax.ShapeDtypeStruct((_B, _N), jnp.float32),
            jax.ShapeDtypeStruct((1, 8), jnp.float32),
        ],
    )(failure_probability, failure_label, failure_timing, ftv, v, th, lf, tl)

    pp = _powerflow(a, c, ei, g, bsus)

    pf_sum = pl.pallas_call(
        _final_kernel,
        out_specs=pl.BlockSpec(memory_space=pltpu.SMEM),
        out_shape=jax.ShapeDtypeStruct((1, 1), jnp.float32),
    )(pp, pinj)[0, 0]

    bce_sum = s[0, 0]
    cnt = s[0, 1]
    sq_sum = s[0, 2]
    stab_sum = s[0, 3]
    cap_sum = s[0, 4]

    nbn = jnp.float32(_B * _N)
    l_pred = -bce_sum / nbn
    l_timing = sq_sum / jnp.maximum(cnt, 1.0)
    l_pred = l_pred + jnp.where(cnt > 0, 0.5 * l_timing, 0.0)
    l_pf = pf_sum / nbn
    l_cap = cap_sum / jnp.float32(_B * _E)
    l_stab = stab_sum / nbn
    l_temporal = jnp.float32(0.0)
    l_total = (l_pred + 0.1 * l_pf + 0.05 * l_cap + 0.05 * l_stab
               + 0.02 * l_temporal)
    return (l_total, l_pred, l_pf, l_cap, l_stab, l_temporal)
